# baseline (speedup 1.0000x reference)
import functools
import jax
import jax.numpy as jnp
from jax import lax
from jax.experimental import pallas as pl
from jax.experimental.pallas import tpu as pltpu
from jax.experimental.pallas import tpu_sc as plsc

N = 10000
E = 320000
D = 128
NC, NS = 2, 16
NW = NC * NS
CH = 64
EPT = 10240
EPAD = NW * EPT
NCHUNK = EPT // CH
NPAD = 10240
NDEG = 10240

_sc_mesh = plsc.VectorSubcoreMesh(
    core_axis_name="c", subcore_axis_name="s", num_cores=NC, num_subcores=NS)


@functools.partial(
    pl.kernel,
    out_type=jax.ShapeDtypeStruct((NC * 2 * NDEG,), jnp.float32),
    mesh=_sc_mesh,
    scratch_types=[
        pltpu.VMEM((CH,), jnp.int32),
        pltpu.VMEM((CH,), jnp.float32),
        pltpu.VMEM((1024,), jnp.float32),
        pltpu.VMEM_SHARED((NDEG,), jnp.float32),
        pltpu.VMEM_SHARED((NDEG,), jnp.float32),
    ],
)
def _degree_kernel(src_hbm, dst_hbm, out_hbm, idx_v, ones_v, zb_v,
                   dego_sh, degi_sh):
    c = lax.axis_index("c")
    s = lax.axis_index("s")
    wid = s * NC + c

    def fill(i, _):
        ones_v[pl.ds(i * 16, 16)] = jnp.full((16,), 1.0, jnp.float32)
        return 0
    lax.fori_loop(0, CH // 16, fill, 0)

    def fill0(i, _):
        zb_v[pl.ds(i * 16, 16)] = jnp.zeros((16,), jnp.float32)
        return 0
    lax.fori_loop(0, 1024 // 16, fill0, 0)

    @pl.when(s < 10)
    def _():
        pltpu.sync_copy(zb_v, dego_sh.at[pl.ds(s * 1024, 1024)])
        pltpu.sync_copy(zb_v, degi_sh.at[pl.ds(s * 1024, 1024)])

    plsc.subcore_barrier()

    base = wid * EPT

    def body(j, _):
        off = base + j * CH
        pltpu.sync_copy(src_hbm.at[pl.ds(off, CH)], idx_v)
        pltpu.sync_copy(ones_v, dego_sh.at[idx_v], add=True)
        pltpu.sync_copy(dst_hbm.at[pl.ds(off, CH)], idx_v)
        pltpu.sync_copy(ones_v, degi_sh.at[idx_v], add=True)
        return 0
    lax.fori_loop(0, NCHUNK, body, 0)

    plsc.subcore_barrier()

    @pl.when(s < 10)
    def _():
        pltpu.sync_copy(dego_sh.at[pl.ds(s * 1024, 1024)],
                        out_hbm.at[pl.ds(c * 2 * NDEG + s * 1024, 1024)])
        pltpu.sync_copy(degi_sh.at[pl.ds(s * 1024, 1024)],
                        out_hbm.at[pl.ds(c * 2 * NDEG + NDEG + s * 1024, 1024)])


NSLOT = 5
NDIST = 3
NSUPC = 16
C0CH = 240
C1CH = 80
NSUP = (16 * (C0CH + C1CH)) // NSUPC


@functools.partial(
    pl.kernel,
    out_type=jax.ShapeDtypeStruct((NC, N, D), jnp.float32),
    mesh=_sc_mesh,
    scratch_types=[
        pltpu.VMEM((2, NSUPC, CH), jnp.int32),
        pltpu.VMEM((2, NSUPC, CH), jnp.int32),
        pltpu.VMEM((NSLOT, CH, D), jnp.float32),
        pltpu.VMEM_SHARED((NPAD, D), jnp.float32),
        [pltpu.SemaphoreType.DMA] * NSLOT,
        [pltpu.SemaphoreType.DMA] * NSLOT,
        pltpu.SemaphoreType.DMA,
    ],
)
def _edge_kernel(h_hbm, src_hbm, dst_hbm, out_hbm,
                 sidx_v, didx_v, rows_v, agg_sh, gsems, ssems, zsem):
    c = lax.axis_index("c")
    s = lax.axis_index("s")

    def fill0(i, _):
        rows_v[0, i // 8, pl.ds((i % 8) * 16, 16)] = jnp.zeros((16,),
                                                               jnp.float32)
        return 0
    lax.fori_loop(0, 32 * 8, fill0, 0)
    zb = rows_v.at[0].at[pl.ds(0, 32)]

    for k in range(20):
        pltpu.async_copy(zb, agg_sh.at[pl.ds(s * 640 + k * 32, 32)], zsem)
    for k in range(20):
        pltpu.make_async_copy(zb, agg_sh.at[pl.ds(0, 32)], zsem).wait()

    plsc.subcore_barrier()

    base_sup = jnp.where(c == 0, s * (C0CH // NSUPC),
                         16 * (C0CH // NSUPC) + s * (C1CH // NSUPC))
    nch = jnp.where(c == 0, C0CH, C1CH)

    def load_super(q):
        pltpu.sync_copy(src_hbm.at[base_sup + q], sidx_v.at[q % 2])
        pltpu.sync_copy(dst_hbm.at[base_sup + q], didx_v.at[q % 2])

    def start_gather(ch, b):
        pltpu.async_copy(
            h_hbm.at[sidx_v.at[(ch // NSUPC) % 2, ch % NSUPC]],
            rows_v.at[b], gsems[b])

    def wait_gather(b):
        pltpu.make_async_copy(h_hbm.at[sidx_v.at[0, 0]], rows_v.at[b],
                              gsems[b]).wait()

    def start_scatter(ch, b):
        pltpu.async_copy(
            rows_v.at[b],
            agg_sh.at[didx_v.at[(ch // NSUPC) % 2, ch % NSUPC]],
            ssems[b], add=True)

    def wait_scatter(b):
        pltpu.make_async_copy(rows_v.at[b], agg_sh.at[didx_v.at[0, 0]],
                              ssems[b]).wait()

    @pl.when(nch > 0)
    def _():
        load_super(0)
        for b in range(NDIST):
            start_gather(b, b)

    def body(i, _):
        for b in range(NSLOT):
            ch = i * NSLOT + b
            p = ch + NDIST
            bp = (b + NDIST) % NSLOT

            @pl.when(jnp.logical_and(p % NSUPC == 0, p < nch))
            def _():
                load_super(p // NSUPC)

            wait_gather(b)
            start_scatter(ch, b)

            @pl.when(jnp.logical_and(p < nch, ch >= NSLOT - NDIST))
            def _():
                wait_scatter(bp)

            @pl.when(p < nch)
            def _():
                start_gather(p, bp)
        return 0
    lax.fori_loop(0, nch // NSLOT, body, 0)

    @pl.when(nch > 0)
    def _():
        for b in range(NSLOT):
            wait_scatter(b)

    plsc.subcore_barrier()

    @pl.when(s < 10)
    def _():
        pltpu.sync_copy(agg_sh.at[pl.ds(s * 1000, 1000)],
                        out_hbm.at[c].at[pl.ds(s * 1000, 1000)])


_RB = 1000


def _make_dense(mode):
    def body(*refs):
        if mode == "first":
            x_ref, deg_ref, w_ref, o_ref = refs
            h = x_ref[...]
        elif mode == "mid":
            s_ref, deg_ref, b_ref, w_ref, o_ref = refs
            sv = s_ref[...]
            h = sv[0] + sv[1]
        else:
            s_ref, deg_ref, b_ref, o_ref = refs
            sv = s_ref[...]
            h = sv[0] + sv[1]
        dv = deg_ref[...]
        if mode != "first":
            nd = lax.rsqrt(jnp.maximum(dv[1], 1.0))
            h = h * nd + b_ref[...]
            if mode == "last":
                o_ref[...] = h
                return
            h = jnp.tanh(h)
        ns = lax.rsqrt(jnp.maximum(dv[0], 1.0))
        h = h * ns
        o_ref[...] = jnp.dot(h, w_ref[...], preferred_element_type=jnp.float32)

    grid = (N // _RB,)
    deg_spec = pl.BlockSpec((2, _RB, 1), lambda i: (0, i, 0))
    b_spec = pl.BlockSpec((1, D), lambda i: (0, 0))
    w_spec = pl.BlockSpec((D, D), lambda i: (0, 0))
    x_spec = pl.BlockSpec((_RB, D), lambda i: (i, 0))
    s_spec = pl.BlockSpec((2, _RB, D), lambda i: (0, i, 0))
    if mode == "first":
        in_specs = [x_spec, deg_spec, w_spec]
    elif mode == "mid":
        in_specs = [s_spec, deg_spec, b_spec, w_spec]
    else:
        in_specs = [s_spec, deg_spec, b_spec]
    return pl.pallas_call(
        body,
        grid=grid,
        in_specs=in_specs,
        out_specs=x_spec,
        out_shape=jax.ShapeDtypeStruct((N, D), jnp.float32),
        compiler_params=pltpu.CompilerParams(
            dimension_semantics=("parallel",)),
    )


_dense_first = _make_dense("first")
_dense_mid = _make_dense("mid")
_dense_last = _make_dense("last")


@jax.jit
def kernel(x, edge_index, W1, b1, W2, b2, W3, b3, W4, b4, W5, b5):
    src = edge_index[0]
    dst = edge_index[1]
    pad = EPAD - E
    dummy = N + (jnp.arange(pad, dtype=jnp.int32) % 16)
    src_deg = jnp.concatenate([src, dummy])
    src_edge = jnp.concatenate([src, jnp.zeros((pad,), jnp.int32)]).reshape(
        NSUP, NSUPC, CH)
    dst_pad = jnp.concatenate([dst, dummy])
    dst_edge = dst_pad.reshape(NSUP, NSUPC, CH)

    degs = _degree_kernel(src_deg, dst_pad)
    deg = degs.reshape(NC, 2, NDEG).sum(axis=0)[:, :N].reshape(2, N, 1)

    h = _dense_first(x, deg, W1)
    for (Wn, bn) in ((W2, b1), (W3, b2), (W4, b3), (W5, b4)):
        sagg = _edge_kernel(h, src_edge, dst_edge)
        h = _dense_mid(sagg, deg, bn.reshape(1, D), Wn)
    sagg = _edge_kernel(h, src_edge, dst_edge)
    return _dense_last(sagg, deg, b5.reshape(1, D))


# speedup vs baseline: 1.2308x; 1.2308x over previous
import functools
import jax
import jax.numpy as jnp
from jax import lax
from jax.experimental import pallas as pl
from jax.experimental.pallas import tpu as pltpu
from jax.experimental.pallas import tpu_sc as plsc

N = 10000
E = 320000
D = 128
NC, NS = 2, 16
NW = NC * NS
CH = 64
EPT = 10240
EPAD = NW * EPT
NCHUNK = EPT // CH
NPAD = 10240
NDEG = 10240

_sc_mesh = plsc.VectorSubcoreMesh(
    core_axis_name="c", subcore_axis_name="s", num_cores=NC, num_subcores=NS)


@functools.partial(
    pl.kernel,
    out_type=jax.ShapeDtypeStruct((NC * 2 * NDEG,), jnp.float32),
    mesh=_sc_mesh,
    scratch_types=[
        pltpu.VMEM((2, CH), jnp.int32),
        pltpu.VMEM((2, CH), jnp.int32),
        pltpu.VMEM((CH,), jnp.float32),
        pltpu.VMEM((1024,), jnp.float32),
        pltpu.VMEM_SHARED((NDEG,), jnp.float32),
        pltpu.VMEM_SHARED((NDEG,), jnp.float32),
        [pltpu.SemaphoreType.DMA] * 2,
        [pltpu.SemaphoreType.DMA] * 2,
        [pltpu.SemaphoreType.DMA] * 2,
        [pltpu.SemaphoreType.DMA] * 2,
    ],
)
def _degree_kernel(src_hbm, dst_hbm, out_hbm, so_v, di_v, ones_v, zb_v,
                   dego_sh, degi_sh, lso, ldi, sso, sdi):
    c = lax.axis_index("c")
    s = lax.axis_index("s")
    wid = s * NC + c

    def fill(i, _):
        ones_v[pl.ds(i * 16, 16)] = jnp.full((16,), 1.0, jnp.float32)
        return 0
    lax.fori_loop(0, CH // 16, fill, 0)

    def fill0(i, _):
        zb_v[pl.ds(i * 16, 16)] = jnp.zeros((16,), jnp.float32)
        return 0
    lax.fori_loop(0, 1024 // 16, fill0, 0)

    @pl.when(s < 10)
    def _():
        pltpu.sync_copy(zb_v, dego_sh.at[pl.ds(s * 1024, 1024)])
        pltpu.sync_copy(zb_v, degi_sh.at[pl.ds(s * 1024, 1024)])

    plsc.subcore_barrier()

    base = wid * EPT

    def load(j, b):
        pltpu.async_copy(src_hbm.at[pl.ds(base + j * CH, CH)], so_v.at[b],
                         lso[b])
        pltpu.async_copy(dst_hbm.at[pl.ds(base + j * CH, CH)], di_v.at[b],
                         ldi[b])

    def wait_load(b):
        pltpu.make_async_copy(src_hbm.at[pl.ds(0, CH)], so_v.at[b],
                              lso[b]).wait()
        pltpu.make_async_copy(dst_hbm.at[pl.ds(0, CH)], di_v.at[b],
                              ldi[b]).wait()

    def scat(b):
        pltpu.async_copy(ones_v, dego_sh.at[so_v.at[b]], sso[b], add=True)
        pltpu.async_copy(ones_v, degi_sh.at[di_v.at[b]], sdi[b], add=True)

    def wait_scat(b):
        pltpu.make_async_copy(ones_v, dego_sh.at[so_v.at[0]], sso[b]).wait()
        pltpu.make_async_copy(ones_v, degi_sh.at[di_v.at[0]], sdi[b]).wait()

    load(0, 0)

    def body(i, _):
        for b in range(2):
            j = i * 2 + b
            nb = 1 - b
            wait_load(b)
            scat(b)

            @pl.when(jnp.logical_and(j + 1 < NCHUNK, j >= 1))
            def _():
                wait_scat(nb)

            @pl.when(j + 1 < NCHUNK)
            def _():
                load(j + 1, nb)
        return 0
    lax.fori_loop(0, NCHUNK // 2, body, 0)

    wait_scat(0)
    wait_scat(1)

    plsc.subcore_barrier()

    @pl.when(s < 10)
    def _():
        pltpu.sync_copy(dego_sh.at[pl.ds(s * 1024, 1024)],
                        out_hbm.at[pl.ds(c * 2 * NDEG + s * 1024, 1024)])
        pltpu.sync_copy(degi_sh.at[pl.ds(s * 1024, 1024)],
                        out_hbm.at[pl.ds(c * 2 * NDEG + NDEG + s * 1024, 1024)])


NSLOT = 4
NDIST = 2
NSUPC = 16
C0CH = 304
C1CH = 16
NSUP = (16 * (C0CH + C1CH)) // NSUPC


@functools.partial(
    pl.kernel,
    out_type=jax.ShapeDtypeStruct((NC, N, D), jnp.float32),
    mesh=_sc_mesh,
    scratch_types=[
        pltpu.VMEM((2, NSUPC, CH), jnp.int32),
        pltpu.VMEM((2, NSUPC, CH), jnp.int32),
        pltpu.VMEM((NSLOT, CH, D), jnp.float32),
        pltpu.VMEM((32, D), jnp.float32),
        pltpu.VMEM_SHARED((NPAD, D), jnp.float32),
        [pltpu.SemaphoreType.DMA] * NSLOT,
        [pltpu.SemaphoreType.DMA] * NSLOT,
        pltpu.SemaphoreType.DMA,
    ],
)
def _edge_kernel(h_hbm, src_hbm, dst_hbm, out_hbm,
                 sidx_v, didx_v, rows_v, zb_v, agg_sh, gsems, ssems, zsem):
    c = lax.axis_index("c")
    s = lax.axis_index("s")

    def fill0(i, _):
        zb_v[i // 8, pl.ds((i % 8) * 16, 16)] = jnp.zeros((16,), jnp.float32)
        return 0
    lax.fori_loop(0, 32 * 8, fill0, 0)

    for k in range(20):
        pltpu.async_copy(zb_v, agg_sh.at[pl.ds(s * 640 + k * 32, 32)], zsem)
    for k in range(20):
        pltpu.make_async_copy(zb_v, agg_sh.at[pl.ds(0, 32)], zsem).wait()

    plsc.subcore_barrier()

    base_sup = jnp.where(c == 0, s * (C0CH // NSUPC),
                         16 * (C0CH // NSUPC) + s * (C1CH // NSUPC))
    nch = jnp.where(c == 0, C0CH, C1CH)

    def load_super(q):
        pltpu.sync_copy(src_hbm.at[base_sup + q], sidx_v.at[q % 2])
        pltpu.sync_copy(dst_hbm.at[base_sup + q], didx_v.at[q % 2])

    def start_gather(ch, b):
        pltpu.async_copy(
            h_hbm.at[sidx_v.at[(ch // NSUPC) % 2, ch % NSUPC]],
            rows_v.at[b], gsems[b])

    def wait_gather(b):
        pltpu.make_async_copy(h_hbm.at[sidx_v.at[0, 0]], rows_v.at[b],
                              gsems[b]).wait()

    def start_scatter(ch, b):
        pltpu.async_copy(
            rows_v.at[b],
            agg_sh.at[didx_v.at[(ch // NSUPC) % 2, ch % NSUPC]],
            ssems[b], add=True)

    def wait_scatter(b):
        pltpu.make_async_copy(rows_v.at[b], agg_sh.at[didx_v.at[0, 0]],
                              ssems[b]).wait()

    @pl.when(nch > 0)
    def _():
        load_super(0)
        for b in range(NDIST):
            start_gather(b, b)

    def body(i, _):
        for b in range(NSLOT):
            ch = i * NSLOT + b
            p = ch + NDIST
            bp = (b + NDIST) % NSLOT

            @pl.when(jnp.logical_and(p % NSUPC == 0, p < nch))
            def _():
                load_super(p // NSUPC)

            wait_gather(b)
            start_scatter(ch, b)

            @pl.when(jnp.logical_and(p < nch, ch >= NSLOT - NDIST))
            def _():
                wait_scatter(bp)

            @pl.when(p < nch)
            def _():
                start_gather(p, bp)
        return 0
    lax.fori_loop(0, nch // NSLOT, body, 0)

    @pl.when(nch > 0)
    def _():
        for b in range(NSLOT):
            wait_scatter(b)

    plsc.subcore_barrier()

    @pl.when(s < 10)
    def _():
        pltpu.sync_copy(agg_sh.at[pl.ds(s * 1000, 1000)],
                        out_hbm.at[c].at[pl.ds(s * 1000, 1000)])


_RB = 1000


def _make_dense(mode):
    def body(*refs):
        if mode == "first":
            x_ref, deg_ref, w_ref, o_ref = refs
            h = x_ref[...]
        elif mode == "mid":
            s_ref, deg_ref, b_ref, w_ref, o_ref = refs
            sv = s_ref[...]
            h = sv[0] + sv[1]
        else:
            s_ref, deg_ref, b_ref, o_ref = refs
            sv = s_ref[...]
            h = sv[0] + sv[1]
        dv = deg_ref[...]
        if mode != "first":
            nd = lax.rsqrt(jnp.maximum(dv[1], 1.0))
            h = h * nd + b_ref[...]
            if mode == "last":
                o_ref[...] = h
                return
            h = jnp.tanh(h)
        ns = lax.rsqrt(jnp.maximum(dv[0], 1.0))
        h = h * ns
        o_ref[...] = jnp.dot(h, w_ref[...], preferred_element_type=jnp.float32)

    grid = (N // _RB,)
    deg_spec = pl.BlockSpec((2, _RB, 1), lambda i: (0, i, 0))
    b_spec = pl.BlockSpec((1, D), lambda i: (0, 0))
    w_spec = pl.BlockSpec((D, D), lambda i: (0, 0))
    x_spec = pl.BlockSpec((_RB, D), lambda i: (i, 0))
    s_spec = pl.BlockSpec((2, _RB, D), lambda i: (0, i, 0))
    if mode == "first":
        in_specs = [x_spec, deg_spec, w_spec]
    elif mode == "mid":
        in_specs = [s_spec, deg_spec, b_spec, w_spec]
    else:
        in_specs = [s_spec, deg_spec, b_spec]
    return pl.pallas_call(
        body,
        grid=grid,
        in_specs=in_specs,
        out_specs=x_spec,
        out_shape=jax.ShapeDtypeStruct((N, D), jnp.float32),
        compiler_params=pltpu.CompilerParams(
            dimension_semantics=("parallel",)),
    )


_dense_first = _make_dense("first")
_dense_mid = _make_dense("mid")
_dense_last = _make_dense("last")


@jax.jit
def kernel(x, edge_index, W1, b1, W2, b2, W3, b3, W4, b4, W5, b5):
    src = edge_index[0]
    dst = edge_index[1]
    pad = EPAD - E
    dummy = N + (jnp.arange(pad, dtype=jnp.int32) % 16)
    src_deg = jnp.concatenate([src, dummy])
    src_edge = jnp.concatenate([src, jnp.zeros((pad,), jnp.int32)]).reshape(
        NSUP, NSUPC, CH)
    dst_pad = jnp.concatenate([dst, dummy])
    dst_edge = dst_pad.reshape(NSUP, NSUPC, CH)

    degs = _degree_kernel(src_deg, dst_pad)
    deg = degs.reshape(NC, 2, NDEG).sum(axis=0)[:, :N].reshape(2, N, 1)

    h = _dense_first(x, deg, W1)
    for (Wn, bn) in ((W2, b1), (W3, b2), (W4, b3), (W5, b4)):
        sagg = _edge_kernel(h, src_edge, dst_edge)
        h = _dense_mid(sagg, deg, bn.reshape(1, D), Wn)
    sagg = _edge_kernel(h, src_edge, dst_edge)
    return _dense_last(sagg, deg, b5.reshape(1, D))
